# baseline (speedup 1.0000x reference)
import functools

import jax
import jax.numpy as jnp
from jax import lax
from jax.experimental import pallas as pl
from jax.experimental.pallas import tpu as pltpu
from jax.experimental.pallas import tpu_sc as plsc

NN = 50000
EE = 800000
HH = 32
NC = 2
NS = 16
LL = 16

CHUNK = 128
RING = 5
EPT = 50560
EPAD = EPT * NS
MBLK = RING * CHUNK
NMBLK = EPT // MBLK
NPAD = 50176
RPT = NPAD // NS
PIECE = 32
NPIECE = RPT // PIECE


def _body(row_hbm, col_hbm, val_hbm, x0_hbm, mean_hbm, y1_hbm, y2_hbm,
          acc, colb, rowb, valb, gidxs, ridxs, vbufs, rows3, zb, m1, m2, m3,
          sg0, sg1, sg2, sg3, sg4, ss0, ss1, ss2, ss3, ss4):
    sem_g = [sg0, sg1, sg2, sg3, sg4]
    sem_s = [ss0, ss1, ss2, ss3, ss4]
    c = lax.axis_index("c")
    s = lax.axis_index("s")
    cN = c * NPAD
    ebase = s * EPT
    rbase = s * RPT

    zero16 = jnp.zeros((LL,), jnp.float32)

    def zfill(i, carry):
        zb[i, pl.ds(0, LL)] = zero16
        zb[i, pl.ds(LL, LL)] = zero16
        return carry
    lax.fori_loop(0, PIECE, zfill, 0)

    def zero_acc():
        def zp(p, carry):
            pltpu.sync_copy(zb, acc.at[pl.ds(rbase + p * PIECE, PIECE)])
            return carry
        lax.fori_loop(0, NPIECE, zp, 0)

    def layer(x_hbm, y_hbm):
        zero_acc()
        plsc.subcore_barrier()

        def mblock(m, carry):
            base = ebase + m * MBLK
            pltpu.sync_copy(col_hbm.at[pl.ds(base, MBLK)], colb)
            pltpu.sync_copy(row_hbm.at[pl.ds(base, MBLK)], rowb)
            pltpu.sync_copy(val_hbm.at[pl.ds(base, MBLK)], valb)

            not_first = m > 0
            if True:
                for j in range(RING):
                    @pl.when(not_first)
                    def _drain(j=j):
                        pltpu.make_async_copy(
                            rows3.at[j], acc.at[ridxs.at[j]],
                            sem_s[j]).wait()
                    off = j * CHUNK
                    for q in range(CHUNK // LL):
                        sl = pl.ds(q * LL, LL)
                        src = pl.ds(off + q * LL, LL)
                        gidxs[j, sl] = colb[src] + cN
                        ridxs[j, sl] = rowb[src]
                        vbufs[j, sl] = valb[src]
                    pltpu.async_copy(x_hbm.at[gidxs.at[j]], rows3.at[j],
                                     sem_g[j])
                for j in range(RING):
                    pltpu.make_async_copy(x_hbm.at[gidxs.at[j]], rows3.at[j],
                                          sem_g[j]).wait()
                    def egroup(g, carry3, j=j):
                        vv = vbufs[j, pl.ds(g * LL, LL)]
                        e0 = g * LL
                        for i in range(LL):
                            v = vv[i]
                            e = e0 + i
                            rows3[j, e, pl.ds(0, LL)] = rows3[j, e, pl.ds(0, LL)] * v
                            rows3[j, e, pl.ds(LL, LL)] = rows3[j, e, pl.ds(LL, LL)] * v
                        return carry3
                    lax.fori_loop(0, CHUNK // LL, egroup, 0)
                    pltpu.async_copy(rows3.at[j], acc.at[ridxs.at[j]],
                                     sem_s[j], add=True)
            return carry
        lax.fori_loop(0, NMBLK, mblock, 0)
        for j in range(RING):
            pltpu.make_async_copy(rows3.at[j], acc.at[ridxs.at[j]],
                                  sem_s[j]).wait()
        plsc.subcore_barrier()

        if y_hbm is not None:
            def cp(p, carry):
                r0 = rbase + p * PIECE
                pltpu.sync_copy(acc.at[pl.ds(r0, PIECE)],
                                y_hbm.at[pl.ds(cN + r0, PIECE)])
                return carry
            lax.fori_loop(0, NPIECE, cp, 0)
            plsc.subcore_barrier()

    layer(x0_hbm, y1_hbm)
    layer(y1_hbm, y2_hbm)
    layer(y2_hbm, None)

    def piece(p, carry):
        r0 = rbase + p * PIECE
        pltpu.sync_copy(x0_hbm.at[pl.ds(cN + r0, PIECE)], m1)
        pltpu.sync_copy(y1_hbm.at[pl.ds(cN + r0, PIECE)], m2)
        pltpu.sync_copy(y2_hbm.at[pl.ds(cN + r0, PIECE)], m3)
        pltpu.sync_copy(acc.at[pl.ds(r0, PIECE)], zb)

        def mrow(i, carry2):
            for off in (0, LL):
                sl = pl.ds(off, LL)
                zb[i, sl] = (zb[i, sl] + m1[i, sl] + m2[i, sl] + m3[i, sl]) * 0.25
            return carry2
        lax.fori_loop(0, PIECE, mrow, 0)
        pltpu.sync_copy(zb, mean_hbm.at[pl.ds(cN + r0, PIECE)])
        return carry
    lax.fori_loop(0, NPIECE, piece, 0)


@functools.partial(jax.jit, static_argnums=())
def _propagate(row, col, vals, x0):
    f32 = jnp.float32
    run = pl.kernel(
        _body,
        out_type=(
            jax.ShapeDtypeStruct((2 * NPAD, HH), f32),
            jax.ShapeDtypeStruct((2 * NPAD, HH), f32),
            jax.ShapeDtypeStruct((2 * NPAD, HH), f32),
        ),
        mesh=plsc.VectorSubcoreMesh(
            core_axis_name="c", subcore_axis_name="s",
            num_cores=NC, num_subcores=NS),
        scratch_types=[
            pltpu.VMEM_SHARED((NPAD, HH), f32),
            pltpu.VMEM((MBLK,), jnp.int32),
            pltpu.VMEM((MBLK,), jnp.int32),
            pltpu.VMEM((MBLK,), f32),
            pltpu.VMEM((RING, CHUNK), jnp.int32),
            pltpu.VMEM((RING, CHUNK), jnp.int32),
            pltpu.VMEM((RING, CHUNK), f32),
            pltpu.VMEM((RING, CHUNK, HH), f32),
            pltpu.VMEM((PIECE, HH), f32),
            pltpu.VMEM((PIECE, HH), f32),
            pltpu.VMEM((PIECE, HH), f32),
            pltpu.VMEM((PIECE, HH), f32),
        ] + [pltpu.SemaphoreType.DMA] * 10,
        compiler_params=pltpu.CompilerParams(use_tc_tiling_on_sc=False),
    )
    return run(row, col, vals, x0)


def kernel(A_indices, A_values, user_emb, item_emb):
    epad = EPAD - EE
    row = jnp.concatenate(
        [A_indices[0].astype(jnp.int32), jnp.full((epad,), NN, jnp.int32)])
    col = jnp.concatenate(
        [A_indices[1].astype(jnp.int32), jnp.zeros((epad,), jnp.int32)])
    vals = jnp.concatenate([A_values, jnp.zeros((epad,), jnp.float32)])
    all_emb = jnp.concatenate([user_emb, item_emb], axis=0)
    pad = jnp.zeros((NPAD - NN, HH), jnp.float32)
    x0 = jnp.concatenate([all_emb[:, :HH], pad, all_emb[:, HH:], pad], axis=0)
    mean_flat, _, _ = _propagate(row, col, vals, x0)
    nu = NN // 2
    user_final = jnp.concatenate(
        [mean_flat[:nu], mean_flat[NPAD:NPAD + nu]], axis=1)
    item_final = jnp.concatenate(
        [mean_flat[nu:NN], mean_flat[NPAD + nu:NPAD + NN]], axis=1)
    return (user_final, item_final)


# speedup vs baseline: 1.1282x; 1.1282x over previous
import functools

import jax
import jax.numpy as jnp
from jax import lax
from jax.experimental import pallas as pl
from jax.experimental.pallas import tpu as pltpu
from jax.experimental.pallas import tpu_sc as plsc

NN = 50000
EE = 800000
HH = 32
NC = 2
NS = 16
LL = 16

CHUNK = 128
RING = 5
MBLK = RING * CHUNK
EPT = 51200
NMBLK = EPT // MBLK
NPAIR = NMBLK // 2
EPAD = EPT * NS
NPAD = 50176
RPT = NPAD // NS
PIECE = 32
NPIECE = RPT // PIECE
WIN = 8


def _body(row_hbm, col_hbm, val_hbm, x0_hbm, mean_hbm, y1_hbm, y2_hbm,
          acc, colb, rowb, valb, gidxs, ridxs, vbufs, rows3, zb, m1, m2, m3,
          sg0, sg1, sg2, sg3, sg4, ss0, ss1, ss2, ss3, ss4, sem_m, sem_z):
    sem_g = [sg0, sg1, sg2, sg3, sg4]
    sem_s = [ss0, ss1, ss2, ss3, ss4]
    c = lax.axis_index("c")
    s = lax.axis_index("s")
    cN = c * NPAD
    ebase = s * EPT
    rbase = s * RPT

    zero16 = jnp.zeros((LL,), jnp.float32)

    def zfill(i, carry):
        zb[i, pl.ds(0, LL)] = zero16
        zb[i, pl.ds(LL, LL)] = zero16
        return carry
    lax.fori_loop(0, PIECE, zfill, 0)

    def zero_acc():
        def zf(p, carry):
            pltpu.async_copy(zb, acc.at[pl.ds(rbase + p * PIECE, PIECE)],
                             sem_z)

            @pl.when(p >= WIN)
            def _():
                pltpu.make_async_copy(
                    zb, acc.at[pl.ds(rbase, PIECE)], sem_z).wait()
            return carry
        lax.fori_loop(0, NPIECE, zf, 0)
        for _ in range(WIN):
            pltpu.make_async_copy(zb, acc.at[pl.ds(rbase, PIECE)],
                                  sem_z).wait()

    meta = ((colb, col_hbm), (rowb, row_hbm), (valb, val_hbm))

    def layer(x_hbm, y_hbm):
        zero_acc()
        plsc.subcore_barrier()

        for ref, hbm in meta:
            pltpu.async_copy(hbm.at[pl.ds(ebase, MBLK)], ref.at[0], sem_m)

        def pairblock(p, carry):
            for d in range(2):
                m = 2 * p + d
                base = ebase + m * MBLK
                for ref, hbm in meta:
                    pltpu.make_async_copy(hbm.at[pl.ds(base, MBLK)],
                                          ref.at[d], sem_m).wait()

                @pl.when(m + 1 < NMBLK)
                def _prefetch(d=d, base=base):
                    for ref, hbm in meta:
                        pltpu.async_copy(hbm.at[pl.ds(base + MBLK, MBLK)],
                                         ref.at[1 - d], sem_m)
                not_first = m > 0
                for j in range(RING):
                    @pl.when(not_first)
                    def _drain(j=j):
                        pltpu.make_async_copy(
                            rows3.at[j], acc.at[ridxs.at[j]],
                            sem_s[j]).wait()
                    off = j * CHUNK
                    for q in range(CHUNK // LL):
                        sl = pl.ds(q * LL, LL)
                        src = pl.ds(off + q * LL, LL)
                        gidxs[j, sl] = colb[d, src] + cN
                        ridxs[j, sl] = rowb[d, src]
                        vbufs[j, sl] = valb[d, src]
                    pltpu.async_copy(x_hbm.at[gidxs.at[j]], rows3.at[j],
                                     sem_g[j])
                for j in range(RING):
                    pltpu.make_async_copy(x_hbm.at[gidxs.at[j]],
                                          rows3.at[j], sem_g[j]).wait()

                    def egroup(g, carry3, j=j):
                        vv = vbufs[j, pl.ds(g * LL, LL)]
                        e0 = g * LL
                        for i in range(LL):
                            v = vv[i]
                            e = e0 + i
                            rows3[j, e, pl.ds(0, LL)] = rows3[j, e, pl.ds(0, LL)] * v
                            rows3[j, e, pl.ds(LL, LL)] = rows3[j, e, pl.ds(LL, LL)] * v
                        return carry3
                    lax.fori_loop(0, CHUNK // LL, egroup, 0)
                    pltpu.async_copy(rows3.at[j], acc.at[ridxs.at[j]],
                                     sem_s[j], add=True)
            return carry
        lax.fori_loop(0, NPAIR, pairblock, 0)
        for j in range(RING):
            pltpu.make_async_copy(rows3.at[j], acc.at[ridxs.at[j]],
                                  sem_s[j]).wait()
        plsc.subcore_barrier()

        if y_hbm is not None:
            def cf(p, carry):
                r0 = rbase + p * PIECE
                pltpu.async_copy(acc.at[pl.ds(r0, PIECE)],
                                 y_hbm.at[pl.ds(cN + r0, PIECE)], sem_z)

                @pl.when(p >= WIN)
                def _():
                    pltpu.make_async_copy(
                        acc.at[pl.ds(rbase, PIECE)],
                        y_hbm.at[pl.ds(cN + rbase, PIECE)], sem_z).wait()
                return carry
            lax.fori_loop(0, NPIECE, cf, 0)
            for _ in range(WIN):
                pltpu.make_async_copy(
                    acc.at[pl.ds(rbase, PIECE)],
                    y_hbm.at[pl.ds(cN + rbase, PIECE)], sem_z).wait()
            plsc.subcore_barrier()

    layer(x0_hbm, y1_hbm)
    layer(y1_hbm, y2_hbm)
    layer(y2_hbm, None)

    def piece(p, carry):
        r0 = rbase + p * PIECE
        pltpu.async_copy(x0_hbm.at[pl.ds(cN + r0, PIECE)], m1, sem_m)
        pltpu.async_copy(y1_hbm.at[pl.ds(cN + r0, PIECE)], m2, sem_m)
        pltpu.async_copy(y2_hbm.at[pl.ds(cN + r0, PIECE)], m3, sem_m)
        pltpu.sync_copy(acc.at[pl.ds(r0, PIECE)], zb)
        for _ in range(3):
            pltpu.make_async_copy(x0_hbm.at[pl.ds(cN + r0, PIECE)], m1,
                                  sem_m).wait()

        def mrow(i, carry2):
            for off in (0, LL):
                sl = pl.ds(off, LL)
                zb[i, sl] = (zb[i, sl] + m1[i, sl] + m2[i, sl] + m3[i, sl]) * 0.25
            return carry2
        lax.fori_loop(0, PIECE, mrow, 0)
        pltpu.sync_copy(zb, mean_hbm.at[pl.ds(cN + r0, PIECE)])
        return carry
    lax.fori_loop(0, NPIECE, piece, 0)


@functools.partial(jax.jit, static_argnums=())
def _propagate(row, col, vals, x0):
    f32 = jnp.float32
    run = pl.kernel(
        _body,
        out_type=(
            jax.ShapeDtypeStruct((2 * NPAD, HH), f32),
            jax.ShapeDtypeStruct((2 * NPAD, HH), f32),
            jax.ShapeDtypeStruct((2 * NPAD, HH), f32),
        ),
        mesh=plsc.VectorSubcoreMesh(
            core_axis_name="c", subcore_axis_name="s",
            num_cores=NC, num_subcores=NS),
        scratch_types=[
            pltpu.VMEM_SHARED((NPAD, HH), f32),
            pltpu.VMEM((2, MBLK), jnp.int32),
            pltpu.VMEM((2, MBLK), jnp.int32),
            pltpu.VMEM((2, MBLK), f32),
            pltpu.VMEM((RING, CHUNK), jnp.int32),
            pltpu.VMEM((RING, CHUNK), jnp.int32),
            pltpu.VMEM((RING, CHUNK), f32),
            pltpu.VMEM((RING, CHUNK, HH), f32),
            pltpu.VMEM((PIECE, HH), f32),
            pltpu.VMEM((PIECE, HH), f32),
            pltpu.VMEM((PIECE, HH), f32),
            pltpu.VMEM((PIECE, HH), f32),
        ] + [pltpu.SemaphoreType.DMA] * 12,
        compiler_params=pltpu.CompilerParams(use_tc_tiling_on_sc=False),
    )
    return run(row, col, vals, x0)


def kernel(A_indices, A_values, user_emb, item_emb):
    epad = EPAD - EE
    row = jnp.concatenate(
        [A_indices[0].astype(jnp.int32), jnp.full((epad,), NN, jnp.int32)])
    col = jnp.concatenate(
        [A_indices[1].astype(jnp.int32), jnp.zeros((epad,), jnp.int32)])
    vals = jnp.concatenate([A_values, jnp.zeros((epad,), jnp.float32)])
    all_emb = jnp.concatenate([user_emb, item_emb], axis=0)
    pad = jnp.zeros((NPAD - NN, HH), jnp.float32)
    x0 = jnp.concatenate([all_emb[:, :HH], pad, all_emb[:, HH:], pad], axis=0)
    mean_flat, _, _ = _propagate(row, col, vals, x0)
    nu = NN // 2
    user_final = jnp.concatenate(
        [mean_flat[:nu], mean_flat[NPAD:NPAD + nu]], axis=1)
    item_final = jnp.concatenate(
        [mean_flat[nu:NN], mean_flat[NPAD + nu:NPAD + NN]], axis=1)
    return (user_final, item_final)


# speedup vs baseline: 1.7737x; 1.5721x over previous
import functools

import jax
import jax.numpy as jnp
from jax import lax
from jax.experimental import pallas as pl
from jax.experimental.pallas import tpu as pltpu
from jax.experimental.pallas import tpu_sc as plsc

NN = 50000
EE = 800000
HH = 32
NC = 2
NS = 16
LL = 16

CHUNK = 80
RING = 5
ITC = RING * CHUNK
MBLK = 2000
EPT = EE // NS
NMBLK = EPT // MBLK
NIT = MBLK // ITC
NPAD = 50176
RPT = NPAD // NS
PIECE = 56
NPIECE = RPT // PIECE
WIN = 8


def _body(row_hbm, col_hbm, val_hbm, x0_hbm, mean_hbm, y1_hbm, y2_hbm,
          acc, colb, rowb, valb, gidxs, ridxs, vbufs, rows3, zb, m1, m2, m3,
          sg0, sg1, sg2, sg3, sg4, ss0, ss1, ss2, ss3, ss4, sem_m, sem_z):
    sem_g = [sg0, sg1, sg2, sg3, sg4]
    sem_s = [ss0, ss1, ss2, ss3, ss4]
    c = lax.axis_index("c")
    s = lax.axis_index("s")
    cN = c * NPAD
    ebase = s * EPT
    rbase = s * RPT

    zero16 = jnp.zeros((LL,), jnp.float32)

    def zfill(i, carry):
        zb[i, pl.ds(0, LL)] = zero16
        zb[i, pl.ds(LL, LL)] = zero16
        return carry
    lax.fori_loop(0, PIECE, zfill, 0)

    def zero_acc():
        def zf(p, carry):
            pltpu.async_copy(zb, acc.at[pl.ds(rbase + p * PIECE, PIECE)],
                             sem_z)

            @pl.when(p >= WIN)
            def _():
                pltpu.make_async_copy(
                    zb, acc.at[pl.ds(rbase, PIECE)], sem_z).wait()
            return carry
        lax.fori_loop(0, NPIECE, zf, 0)
        for _ in range(WIN):
            pltpu.make_async_copy(zb, acc.at[pl.ds(rbase, PIECE)],
                                  sem_z).wait()

    meta = ((colb, col_hbm), (rowb, row_hbm), (valb, val_hbm))

    def layer(x_hbm, y_hbm):
        zero_acc()
        plsc.subcore_barrier()

        def mblock(m, carry):
            base = ebase + m * MBLK
            for ref, hbm in meta:
                pltpu.async_copy(hbm.at[pl.ds(base, MBLK)], ref, sem_m)
            for ref, hbm in meta:
                pltpu.make_async_copy(hbm.at[pl.ds(base, MBLK)], ref,
                                      sem_m).wait()

            def ringpass(t, carry2):
                off0 = t * ITC
                not_first = (m + t) > 0
                for j in range(RING):
                    @pl.when(not_first)
                    def _drain(j=j):
                        pltpu.make_async_copy(
                            rows3.at[j], acc.at[ridxs.at[j]],
                            sem_s[j]).wait()
                    off = off0 + j * CHUNK
                    for q in range(CHUNK // LL):
                        sl = pl.ds(q * LL, LL)
                        src = pl.ds(off + q * LL, LL)
                        gidxs[j, sl] = colb[src] + cN
                        ridxs[j, sl] = rowb[src]
                        vbufs[j, sl] = valb[src]
                    pltpu.async_copy(x_hbm.at[gidxs.at[j]], rows3.at[j],
                                     sem_g[j])
                for j in range(RING):
                    pltpu.make_async_copy(x_hbm.at[gidxs.at[j]],
                                          rows3.at[j], sem_g[j]).wait()

                    def egroup(g, carry3, j=j):
                        vv = vbufs[j, pl.ds(g * LL, LL)]
                        e0 = g * LL
                        for i in range(LL):
                            v = vv[i]
                            e = e0 + i
                            rows3[j, e, pl.ds(0, LL)] = rows3[j, e, pl.ds(0, LL)] * v
                            rows3[j, e, pl.ds(LL, LL)] = rows3[j, e, pl.ds(LL, LL)] * v
                        return carry3
                    lax.fori_loop(0, CHUNK // LL, egroup, 0)
                    pltpu.async_copy(rows3.at[j], acc.at[ridxs.at[j]],
                                     sem_s[j], add=True)
                return carry2
            lax.fori_loop(0, NIT, ringpass, 0)
            return carry
        lax.fori_loop(0, NMBLK, mblock, 0)
        for j in range(RING):
            pltpu.make_async_copy(rows3.at[j], acc.at[ridxs.at[j]],
                                  sem_s[j]).wait()
        plsc.subcore_barrier()

        if y_hbm is not None:
            def cf(p, carry):
                r0 = rbase + p * PIECE
                pltpu.async_copy(acc.at[pl.ds(r0, PIECE)],
                                 y_hbm.at[pl.ds(cN + r0, PIECE)], sem_z)

                @pl.when(p >= WIN)
                def _():
                    pltpu.make_async_copy(
                        acc.at[pl.ds(rbase, PIECE)],
                        y_hbm.at[pl.ds(cN + rbase, PIECE)], sem_z).wait()
                return carry
            lax.fori_loop(0, NPIECE, cf, 0)
            for _ in range(WIN):
                pltpu.make_async_copy(
                    acc.at[pl.ds(rbase, PIECE)],
                    y_hbm.at[pl.ds(cN + rbase, PIECE)], sem_z).wait()
            plsc.subcore_barrier()

    layer(x0_hbm, y1_hbm)
    layer(y1_hbm, y2_hbm)
    layer(y2_hbm, None)

    def piece(p, carry):
        r0 = rbase + p * PIECE
        pltpu.async_copy(x0_hbm.at[pl.ds(cN + r0, PIECE)], m1, sem_m)
        pltpu.async_copy(y1_hbm.at[pl.ds(cN + r0, PIECE)], m2, sem_m)
        pltpu.async_copy(y2_hbm.at[pl.ds(cN + r0, PIECE)], m3, sem_m)
        pltpu.sync_copy(acc.at[pl.ds(r0, PIECE)], zb)
        for _ in range(3):
            pltpu.make_async_copy(x0_hbm.at[pl.ds(cN + r0, PIECE)], m1,
                                  sem_m).wait()

        def mrow(i, carry2):
            for off in (0, LL):
                sl = pl.ds(off, LL)
                zb[i, sl] = (zb[i, sl] + m1[i, sl] + m2[i, sl] + m3[i, sl]) * 0.25
            return carry2
        lax.fori_loop(0, PIECE, mrow, 0)
        pltpu.sync_copy(zb, mean_hbm.at[pl.ds(cN + r0, PIECE)])
        return carry
    lax.fori_loop(0, NPIECE, piece, 0)


@functools.partial(jax.jit, static_argnums=())
def _propagate(row, col, vals, x0):
    f32 = jnp.float32
    run = pl.kernel(
        _body,
        out_type=(
            jax.ShapeDtypeStruct((2 * NPAD, HH), f32),
            jax.ShapeDtypeStruct((2 * NPAD, HH), f32),
            jax.ShapeDtypeStruct((2 * NPAD, HH), f32),
        ),
        mesh=plsc.VectorSubcoreMesh(
            core_axis_name="c", subcore_axis_name="s",
            num_cores=NC, num_subcores=NS),
        scratch_types=[
            pltpu.VMEM_SHARED((NPAD, HH), f32),
            pltpu.VMEM((MBLK,), jnp.int32),
            pltpu.VMEM((MBLK,), jnp.int32),
            pltpu.VMEM((MBLK,), f32),
            pltpu.VMEM((RING, CHUNK), jnp.int32),
            pltpu.VMEM((RING, CHUNK), jnp.int32),
            pltpu.VMEM((RING, CHUNK), f32),
            pltpu.VMEM((RING, CHUNK, HH), f32),
            pltpu.VMEM((PIECE, HH), f32),
            pltpu.VMEM((PIECE, HH), f32),
            pltpu.VMEM((PIECE, HH), f32),
            pltpu.VMEM((PIECE, HH), f32),
        ] + [pltpu.SemaphoreType.DMA] * 12,
        compiler_params=pltpu.CompilerParams(use_tc_tiling_on_sc=False),
    )
    return run(row, col, vals, x0)


def kernel(A_indices, A_values, user_emb, item_emb):
    row = A_indices[0].astype(jnp.int32)
    col = A_indices[1].astype(jnp.int32)
    vals = A_values
    all_emb = jnp.concatenate([user_emb, item_emb], axis=0)
    pad = jnp.zeros((NPAD - NN, HH), jnp.float32)
    x0 = jnp.concatenate([all_emb[:, :HH], pad, all_emb[:, HH:], pad], axis=0)
    mean_flat, _, _ = _propagate(row, col, vals, x0)
    nu = NN // 2
    user_final = jnp.concatenate(
        [mean_flat[:nu], mean_flat[NPAD:NPAD + nu]], axis=1)
    item_final = jnp.concatenate(
        [mean_flat[nu:NN], mean_flat[NPAD + nu:NPAD + NN]], axis=1)
    return (user_final, item_final)


# speedup vs baseline: 1.8446x; 1.0399x over previous
import functools

import jax
import jax.numpy as jnp
from jax import lax
from jax.experimental import pallas as pl
from jax.experimental.pallas import tpu as pltpu
from jax.experimental.pallas import tpu_sc as plsc

NN = 50000
EE = 800000
HH = 32
NC = 2
NS = 16
LL = 16

CHUNK = 80
RING = 5
ITC = RING * CHUNK
MBLK = 2000
EPT = EE // NS
NMBLK = EPT // MBLK
NIT = MBLK // ITC
NPAD = 50176
RPT = NPAD // NS
PIECE = 56
NPIECE = RPT // PIECE
WIN = 8


def _body(row_hbm, col_hbm, val_hbm, x0_hbm, mean_hbm, y1_hbm, y2_hbm,
          acc, colb, rowb, valb, gidxs, ridxs, vbufs, rows3, zb, m1, m2, m3,
          sg0, sg1, sg2, sg3, sg4, ss0, ss1, ss2, ss3, ss4, sem_m, sem_z):
    sem_g = [sg0, sg1, sg2, sg3, sg4]
    sem_s = [ss0, ss1, ss2, ss3, ss4]
    c = lax.axis_index("c")
    s = lax.axis_index("s")
    cN = c * NPAD
    ebase = s * EPT
    rbase = s * RPT

    zero16 = jnp.zeros((LL,), jnp.float32)

    def zfill(i, carry):
        zb[i, pl.ds(0, LL)] = zero16
        zb[i, pl.ds(LL, LL)] = zero16
        return carry
    lax.fori_loop(0, PIECE, zfill, 0)

    def zero_acc():
        def zf(p, carry):
            pltpu.async_copy(zb, acc.at[pl.ds(rbase + p * PIECE, PIECE)],
                             sem_z)

            @pl.when(p >= WIN)
            def _():
                pltpu.make_async_copy(
                    zb, acc.at[pl.ds(rbase, PIECE)], sem_z).wait()
            return carry
        lax.fori_loop(0, NPIECE, zf, 0)
        for _ in range(WIN):
            pltpu.make_async_copy(zb, acc.at[pl.ds(rbase, PIECE)],
                                  sem_z).wait()

    meta = ((colb, col_hbm), (rowb, row_hbm), (valb, val_hbm))

    def layer(x_hbm, y_hbm):
        zero_acc()
        plsc.subcore_barrier()

        def mblock(m, carry):
            base = ebase + m * MBLK
            for ref, hbm in meta:
                pltpu.async_copy(hbm.at[pl.ds(base, MBLK)], ref, sem_m)
            for ref, hbm in meta:
                pltpu.make_async_copy(hbm.at[pl.ds(base, MBLK)], ref,
                                      sem_m).wait()

            def ringpass(t, carry2):
                off0 = t * ITC
                not_first = (m + t) > 0
                for j in range(RING):
                    @pl.when(not_first)
                    def _drain(j=j):
                        pltpu.make_async_copy(
                            rows3.at[j], acc.at[ridxs.at[j]],
                            sem_s[j]).wait()
                    off = off0 + j * CHUNK
                    for q in range(CHUNK // LL):
                        sl = pl.ds(q * LL, LL)
                        src = pl.ds(off + q * LL, LL)
                        gidxs[j, sl] = colb[src] + cN
                        ridxs[j, sl] = rowb[src]
                        vbufs[j, sl] = valb[src]
                    pltpu.async_copy(x_hbm.at[gidxs.at[j]], rows3.at[j],
                                     sem_g[j])
                for j in range(RING):
                    pltpu.make_async_copy(x_hbm.at[gidxs.at[j]],
                                          rows3.at[j], sem_g[j]).wait()

                    for g in range(CHUNK // LL):
                        vv = vbufs[j, pl.ds(g * LL, LL)]
                        e0 = g * LL
                        for i in range(LL):
                            v = vv[i]
                            e = e0 + i
                            rows3[j, e, pl.ds(0, LL)] = rows3[j, e, pl.ds(0, LL)] * v
                            rows3[j, e, pl.ds(LL, LL)] = rows3[j, e, pl.ds(LL, LL)] * v
                    pltpu.async_copy(rows3.at[j], acc.at[ridxs.at[j]],
                                     sem_s[j], add=True)
                return carry2
            lax.fori_loop(0, NIT, ringpass, 0)
            return carry
        lax.fori_loop(0, NMBLK, mblock, 0)
        for j in range(RING):
            pltpu.make_async_copy(rows3.at[j], acc.at[ridxs.at[j]],
                                  sem_s[j]).wait()
        plsc.subcore_barrier()

        if y_hbm is not None:
            def cf(p, carry):
                r0 = rbase + p * PIECE
                pltpu.async_copy(acc.at[pl.ds(r0, PIECE)],
                                 y_hbm.at[pl.ds(cN + r0, PIECE)], sem_z)

                @pl.when(p >= WIN)
                def _():
                    pltpu.make_async_copy(
                        acc.at[pl.ds(rbase, PIECE)],
                        y_hbm.at[pl.ds(cN + rbase, PIECE)], sem_z).wait()
                return carry
            lax.fori_loop(0, NPIECE, cf, 0)
            for _ in range(WIN):
                pltpu.make_async_copy(
                    acc.at[pl.ds(rbase, PIECE)],
                    y_hbm.at[pl.ds(cN + rbase, PIECE)], sem_z).wait()
            plsc.subcore_barrier()

    layer(x0_hbm, y1_hbm)
    layer(y1_hbm, y2_hbm)
    layer(y2_hbm, None)

    def piece(p, carry):
        r0 = rbase + p * PIECE
        pltpu.async_copy(x0_hbm.at[pl.ds(cN + r0, PIECE)], m1, sem_m)
        pltpu.async_copy(y1_hbm.at[pl.ds(cN + r0, PIECE)], m2, sem_m)
        pltpu.async_copy(y2_hbm.at[pl.ds(cN + r0, PIECE)], m3, sem_m)
        pltpu.sync_copy(acc.at[pl.ds(r0, PIECE)], zb)
        for _ in range(3):
            pltpu.make_async_copy(x0_hbm.at[pl.ds(cN + r0, PIECE)], m1,
                                  sem_m).wait()

        def mrow(i, carry2):
            for off in (0, LL):
                sl = pl.ds(off, LL)
                zb[i, sl] = (zb[i, sl] + m1[i, sl] + m2[i, sl] + m3[i, sl]) * 0.25
            return carry2
        lax.fori_loop(0, PIECE, mrow, 0)
        pltpu.sync_copy(zb, mean_hbm.at[pl.ds(cN + r0, PIECE)])
        return carry
    lax.fori_loop(0, NPIECE, piece, 0)


@functools.partial(jax.jit, static_argnums=())
def _propagate(row, col, vals, x0):
    f32 = jnp.float32
    run = pl.kernel(
        _body,
        out_type=(
            jax.ShapeDtypeStruct((2 * NPAD, HH), f32),
            jax.ShapeDtypeStruct((2 * NPAD, HH), f32),
            jax.ShapeDtypeStruct((2 * NPAD, HH), f32),
        ),
        mesh=plsc.VectorSubcoreMesh(
            core_axis_name="c", subcore_axis_name="s",
            num_cores=NC, num_subcores=NS),
        scratch_types=[
            pltpu.VMEM_SHARED((NPAD, HH), f32),
            pltpu.VMEM((MBLK,), jnp.int32),
            pltpu.VMEM((MBLK,), jnp.int32),
            pltpu.VMEM((MBLK,), f32),
            pltpu.VMEM((RING, CHUNK), jnp.int32),
            pltpu.VMEM((RING, CHUNK), jnp.int32),
            pltpu.VMEM((RING, CHUNK), f32),
            pltpu.VMEM((RING, CHUNK, HH), f32),
            pltpu.VMEM((PIECE, HH), f32),
            pltpu.VMEM((PIECE, HH), f32),
            pltpu.VMEM((PIECE, HH), f32),
            pltpu.VMEM((PIECE, HH), f32),
        ] + [pltpu.SemaphoreType.DMA] * 12,
        compiler_params=pltpu.CompilerParams(use_tc_tiling_on_sc=False),
    )
    return run(row, col, vals, x0)


def kernel(A_indices, A_values, user_emb, item_emb):
    row = A_indices[0].astype(jnp.int32)
    col = A_indices[1].astype(jnp.int32)
    vals = A_values
    all_emb = jnp.concatenate([user_emb, item_emb], axis=0)
    pad = jnp.zeros((NPAD - NN, HH), jnp.float32)
    x0 = jnp.concatenate([all_emb[:, :HH], pad, all_emb[:, HH:], pad], axis=0)
    mean_flat, _, _ = _propagate(row, col, vals, x0)
    nu = NN // 2
    user_final = jnp.concatenate(
        [mean_flat[:nu], mean_flat[NPAD:NPAD + nu]], axis=1)
    item_final = jnp.concatenate(
        [mean_flat[nu:NN], mean_flat[NPAD + nu:NPAD + NN]], axis=1)
    return (user_final, item_final)


# speedup vs baseline: 2.0051x; 1.0871x over previous
import functools

import jax
import jax.numpy as jnp
from jax import lax
from jax.experimental import pallas as pl
from jax.experimental.pallas import tpu as pltpu
from jax.experimental.pallas import tpu_sc as plsc

NN = 50000
EE = 800000
HH = 32
NC = 2
NS = 16
LL = 16
NU = 25000

CHUNK = 80
RING = 5
ITC = RING * CHUNK
MBLK = 2000
EPT = EE // NS
NMBLK = EPT // MBLK
NIT = MBLK // ITC
NPAD = 50176
RPT = NPAD // NS
PIECE = 56
NPIECE = RPT // PIECE
WIN = 8


def _body(row_hbm, col_hbm, val_hbm, x0_hbm, user_hbm, item_hbm, y1_hbm, y2_hbm,
          acc, colb, rowb, valb, gidxs, ridxs, vbufs, rows3, zb, m1, m2, m3,
          sg0, sg1, sg2, sg3, sg4, ss0, ss1, ss2, ss3, ss4, sem_m, sem_z):
    sem_g = [sg0, sg1, sg2, sg3, sg4]
    sem_s = [ss0, ss1, ss2, ss3, ss4]
    c = lax.axis_index("c")
    s = lax.axis_index("s")
    cN = c * NPAD
    ebase = s * EPT
    rbase = s * RPT

    zero16 = jnp.zeros((LL,), jnp.float32)

    def zfill(i, carry):
        zb[i, pl.ds(0, LL)] = zero16
        zb[i, pl.ds(LL, LL)] = zero16
        return carry
    lax.fori_loop(0, PIECE, zfill, 0)

    def zero_acc():
        def zf(p, carry):
            pltpu.async_copy(zb, acc.at[pl.ds(rbase + p * PIECE, PIECE)],
                             sem_z)

            @pl.when(p >= WIN)
            def _():
                pltpu.make_async_copy(
                    zb, acc.at[pl.ds(rbase, PIECE)], sem_z).wait()
            return carry
        lax.fori_loop(0, NPIECE, zf, 0)
        for _ in range(WIN):
            pltpu.make_async_copy(zb, acc.at[pl.ds(rbase, PIECE)],
                                  sem_z).wait()

    meta = ((colb, col_hbm), (rowb, row_hbm), (valb, val_hbm))

    def layer(x_hbm, y_hbm):
        zero_acc()
        plsc.subcore_barrier()

        def mblock(m, carry):
            base = ebase + m * MBLK
            for ref, hbm in meta:
                pltpu.async_copy(hbm.at[pl.ds(base, MBLK)], ref, sem_m)
            for ref, hbm in meta:
                pltpu.make_async_copy(hbm.at[pl.ds(base, MBLK)], ref,
                                      sem_m).wait()

            def ringpass(t, carry2):
                off0 = t * ITC
                not_first = (m + t) > 0
                for j in range(RING):
                    @pl.when(not_first)
                    def _drain(j=j):
                        pltpu.make_async_copy(
                            rows3.at[j], acc.at[ridxs.at[j]],
                            sem_s[j]).wait()
                    off = off0 + j * CHUNK
                    for q in range(CHUNK // LL):
                        sl = pl.ds(q * LL, LL)
                        src = pl.ds(off + q * LL, LL)
                        gidxs[j, sl] = colb[src] + cN
                        ridxs[j, sl] = rowb[src]
                        vbufs[j, sl] = valb[src]
                    pltpu.async_copy(x_hbm.at[gidxs.at[j]], rows3.at[j],
                                     sem_g[j])
                for j in range(RING):
                    pltpu.make_async_copy(x_hbm.at[gidxs.at[j]],
                                          rows3.at[j], sem_g[j]).wait()

                    for g in range(CHUNK // LL):
                        vv = vbufs[j, pl.ds(g * LL, LL)]
                        e0 = g * LL
                        for i in range(LL):
                            v = vv[i]
                            e = e0 + i
                            rows3[j, e, pl.ds(0, LL)] = rows3[j, e, pl.ds(0, LL)] * v
                            rows3[j, e, pl.ds(LL, LL)] = rows3[j, e, pl.ds(LL, LL)] * v
                    pltpu.async_copy(rows3.at[j], acc.at[ridxs.at[j]],
                                     sem_s[j], add=True)
                return carry2
            lax.fori_loop(0, NIT, ringpass, 0)
            return carry
        lax.fori_loop(0, NMBLK, mblock, 0)
        for j in range(RING):
            pltpu.make_async_copy(rows3.at[j], acc.at[ridxs.at[j]],
                                  sem_s[j]).wait()
        plsc.subcore_barrier()

        if y_hbm is not None:
            def cf(p, carry):
                r0 = rbase + p * PIECE
                pltpu.async_copy(acc.at[pl.ds(r0, PIECE)],
                                 y_hbm.at[pl.ds(cN + r0, PIECE)], sem_z)

                @pl.when(p >= WIN)
                def _():
                    pltpu.make_async_copy(
                        acc.at[pl.ds(rbase, PIECE)],
                        y_hbm.at[pl.ds(cN + rbase, PIECE)], sem_z).wait()
                return carry
            lax.fori_loop(0, NPIECE, cf, 0)
            for _ in range(WIN):
                pltpu.make_async_copy(
                    acc.at[pl.ds(rbase, PIECE)],
                    y_hbm.at[pl.ds(cN + rbase, PIECE)], sem_z).wait()
            plsc.subcore_barrier()

    layer(x0_hbm, y1_hbm)
    layer(y1_hbm, y2_hbm)
    layer(y2_hbm, None)

    def piece(p, carry):
        r0 = rbase + p * PIECE
        pltpu.async_copy(x0_hbm.at[pl.ds(cN + r0, PIECE)], m1, sem_m)
        pltpu.async_copy(y1_hbm.at[pl.ds(cN + r0, PIECE)], m2, sem_m)
        pltpu.async_copy(y2_hbm.at[pl.ds(cN + r0, PIECE)], m3, sem_m)
        pltpu.sync_copy(acc.at[pl.ds(r0, PIECE)], zb)
        for _ in range(3):
            pltpu.make_async_copy(x0_hbm.at[pl.ds(cN + r0, PIECE)], m1,
                                  sem_m).wait()

        def mrow(i, carry2):
            for off in (0, LL):
                sl = pl.ds(off, LL)
                zb[i, sl] = (zb[i, sl] + m1[i, sl] + m2[i, sl] + m3[i, sl]) * 0.25
            return carry2
        lax.fori_loop(0, PIECE, mrow, 0)
        csl = pl.ds(c * HH, HH)

        @pl.when(r0 + PIECE <= NU)
        def _():
            pltpu.sync_copy(zb, user_hbm.at[pl.ds(r0, PIECE), csl])

        @pl.when((r0 >= NU) & (r0 + PIECE <= NN))
        def _():
            pltpu.sync_copy(zb, item_hbm.at[pl.ds(r0 - NU, PIECE), csl])

        @pl.when(r0 == NU - 24)
        def _():
            pltpu.sync_copy(zb.at[pl.ds(0, 24)],
                            user_hbm.at[pl.ds(NU - 24, 24), csl])
            pltpu.sync_copy(zb.at[pl.ds(24, 32)],
                            item_hbm.at[pl.ds(0, 32), csl])

        @pl.when(r0 == NN - 48)
        def _():
            pltpu.sync_copy(zb.at[pl.ds(0, 48)],
                            item_hbm.at[pl.ds(NU - 48, 48), csl])
        return carry
    lax.fori_loop(0, NPIECE, piece, 0)


@functools.partial(jax.jit, static_argnums=())
def _propagate(row, col, vals, x0):
    f32 = jnp.float32
    run = pl.kernel(
        _body,
        out_type=(
            jax.ShapeDtypeStruct((NU, 2 * HH), f32),
            jax.ShapeDtypeStruct((NU, 2 * HH), f32),
            jax.ShapeDtypeStruct((2 * NPAD, HH), f32),
            jax.ShapeDtypeStruct((2 * NPAD, HH), f32),
        ),
        mesh=plsc.VectorSubcoreMesh(
            core_axis_name="c", subcore_axis_name="s",
            num_cores=NC, num_subcores=NS),
        scratch_types=[
            pltpu.VMEM_SHARED((NPAD, HH), f32),
            pltpu.VMEM((MBLK,), jnp.int32),
            pltpu.VMEM((MBLK,), jnp.int32),
            pltpu.VMEM((MBLK,), f32),
            pltpu.VMEM((RING, CHUNK), jnp.int32),
            pltpu.VMEM((RING, CHUNK), jnp.int32),
            pltpu.VMEM((RING, CHUNK), f32),
            pltpu.VMEM((RING, CHUNK, HH), f32),
            pltpu.VMEM((PIECE, HH), f32),
            pltpu.VMEM((PIECE, HH), f32),
            pltpu.VMEM((PIECE, HH), f32),
            pltpu.VMEM((PIECE, HH), f32),
        ] + [pltpu.SemaphoreType.DMA] * 12,
        compiler_params=pltpu.CompilerParams(use_tc_tiling_on_sc=False),
    )
    return run(row, col, vals, x0)


def kernel(A_indices, A_values, user_emb, item_emb):
    row = A_indices[0].astype(jnp.int32)
    col = A_indices[1].astype(jnp.int32)
    vals = A_values
    all_emb = jnp.concatenate([user_emb, item_emb], axis=0)
    pad = jnp.zeros((NPAD - NN, HH), jnp.float32)
    x0 = jnp.concatenate([all_emb[:, :HH], pad, all_emb[:, HH:], pad], axis=0)
    user_final, item_final, _, _ = _propagate(row, col, vals, x0)
    return (user_final, item_final)
